# Initial kernel scaffold; baseline (speedup 1.0000x reference)
import jax
import jax.numpy as jnp
from jax.experimental import pallas as pl


def kernel(x, edge_index, W1, b1, W2, b2):
    raise NotImplementedError("write your pallas kernel here")


# speedup vs baseline: 7.0665x; 7.0665x over previous
import functools
import jax
import jax.numpy as jnp
from jax import lax
from jax.experimental import pallas as pl
from jax.experimental.pallas import tpu as pltpu
from jax.experimental.pallas import tpu_sc as plsc

N = 10000
NP = 10240
NPR = NP // 128
E = 160000
D0, D1, D2 = 256, 256, 128
NC, NS = 2, 16
RT = NP // NS
K1 = 80
K2 = 40
KD = 40
EPT1 = E // NS
EPT2 = E // (NC * NS)
EPTD = E // (NC * NS)

_MESH = plsc.VectorSubcoreMesh(core_axis_name="c", subcore_axis_name="s",
                               num_cores=NC, num_subcores=NS)


def _deg_body(dst_hbm, zeros_k, ones_k, p0, p1, dst_v, rows_v, acc):
  cid = lax.axis_index("c")
  sid = lax.axis_index("s")
  r0 = sid * RT

  pltpu.sync_copy(zeros_k, rows_v)

  def init_chunk(j, carry):
    pltpu.sync_copy(rows_v, acc.at[pl.ds(r0 + j * KD, KD)])
    return carry

  lax.fori_loop(0, RT // KD, init_chunk, 0)
  pltpu.sync_copy(ones_k, rows_v)
  plsc.subcore_barrier()

  def chunk(j, carry):
    base = cid * (E // NC) + sid * EPTD + j * KD
    pltpu.sync_copy(dst_hbm.at[pl.ds(base, KD)], dst_v)
    pltpu.sync_copy(rows_v, acc.at[dst_v], add=True)
    return carry

  lax.fori_loop(0, EPTD // KD, chunk, 0)
  plsc.subcore_barrier()

  def out_chunk(j, carry):
    sl = pl.ds(r0 + j * KD, KD)
    pltpu.sync_copy(acc.at[sl], rows_v)

    @pl.when(cid == 0)
    def _():
      pltpu.sync_copy(rows_v, p0.at[sl])

    @pl.when(cid == 1)
    def _():
      pltpu.sync_copy(rows_v, p1.at[sl])

    return carry

  lax.fori_loop(0, RT // KD, out_chunk, 0)


def _deg_counts(dst):
  zeros_k = jnp.zeros((KD, 128), jnp.float32)
  ones_k = jnp.ones((KD, 128), jnp.float32)
  f = pl.kernel(
      _deg_body,
      out_type=(jax.ShapeDtypeStruct((NP, 128), jnp.float32),
                jax.ShapeDtypeStruct((NP, 128), jnp.float32)),
      mesh=_MESH,
      scratch_types=[
          pltpu.VMEM((KD,), jnp.int32),
          pltpu.VMEM((KD, 128), jnp.float32),
          pltpu.VMEM_SHARED((NP, 128), jnp.float32),
      ],
  )
  return f(dst, zeros_k, ones_k)


def _msg1_body(ya, yb, src_hbm, dst_hbm, oa, ob, src_v, dst_v, rows_v, acc,
               sem):
  cid = lax.axis_index("c")
  sid = lax.axis_index("s")
  r0 = sid * RT

  def init_chunk(j, carry):
    sl = pl.ds(r0 + j * K1, K1)

    @pl.when(cid == 0)
    def _():
      pltpu.sync_copy(ya.at[sl], rows_v)

    @pl.when(cid == 1)
    def _():
      pltpu.sync_copy(yb.at[sl], rows_v)

    pltpu.sync_copy(rows_v, acc.at[sl])
    return carry

  lax.fori_loop(0, RT // K1, init_chunk, 0)
  plsc.subcore_barrier()

  def chunk(j, carry):
    base = sid * EPT1 + j * K1
    pltpu.sync_copy(src_hbm.at[pl.ds(base, K1)], src_v)
    pltpu.sync_copy(dst_hbm.at[pl.ds(base, K1)], dst_v)

    @pl.when(cid == 0)
    def _():
      pltpu.async_copy(ya.at[src_v], rows_v, sem).wait()

    @pl.when(cid == 1)
    def _():
      pltpu.async_copy(yb.at[src_v], rows_v, sem).wait()

    pltpu.sync_copy(rows_v, acc.at[dst_v], add=True)
    return carry

  lax.fori_loop(0, EPT1 // K1, chunk, 0)
  plsc.subcore_barrier()

  def out_chunk(j, carry):
    sl = pl.ds(r0 + j * K1, K1)
    pltpu.sync_copy(acc.at[sl], rows_v)

    @pl.when(cid == 0)
    def _():
      pltpu.sync_copy(rows_v, oa.at[sl])

    @pl.when(cid == 1)
    def _():
      pltpu.sync_copy(rows_v, ob.at[sl])

    return carry

  lax.fori_loop(0, RT // K1, out_chunk, 0)


def _message_pass1(ya, yb, src, dst):
  f = pl.kernel(
      _msg1_body,
      out_type=(jax.ShapeDtypeStruct((NP, 128), jnp.float32),
                jax.ShapeDtypeStruct((NP, 128), jnp.float32)),
      mesh=_MESH,
      scratch_types=[
          pltpu.VMEM((K1,), jnp.int32),
          pltpu.VMEM((K1,), jnp.int32),
          pltpu.VMEM((K1, 128), jnp.float32),
          pltpu.VMEM_SHARED((NP, 128), jnp.float32),
          pltpu.SemaphoreType.DMA,
      ],
  )
  return f(ya, yb, src, dst)


def _msg2_body(y2, src_hbm, dst_hbm, zeros_k, p0, p1, src_v, dst_v, rows_v,
               acc, sem):
  cid = lax.axis_index("c")
  sid = lax.axis_index("s")
  r0 = sid * RT

  def init_chunk(j, carry):
    sl = pl.ds(r0 + j * K2, K2)

    @pl.when(cid == 0)
    def _():
      pltpu.sync_copy(y2.at[sl], rows_v)

    @pl.when(cid == 1)
    def _():
      pltpu.sync_copy(zeros_k, rows_v)

    pltpu.sync_copy(rows_v, acc.at[sl])
    return carry

  lax.fori_loop(0, RT // K2, init_chunk, 0)
  plsc.subcore_barrier()

  def chunk(j, carry):
    base = cid * (E // NC) + sid * EPT2 + j * K2
    pltpu.sync_copy(src_hbm.at[pl.ds(base, K2)], src_v)
    pltpu.sync_copy(dst_hbm.at[pl.ds(base, K2)], dst_v)
    pltpu.async_copy(y2.at[src_v], rows_v, sem).wait()
    pltpu.sync_copy(rows_v, acc.at[dst_v], add=True)
    return carry

  lax.fori_loop(0, EPT2 // K2, chunk, 0)
  plsc.subcore_barrier()

  def out_chunk(j, carry):
    sl = pl.ds(r0 + j * K2, K2)
    pltpu.sync_copy(acc.at[sl], rows_v)

    @pl.when(cid == 0)
    def _():
      pltpu.sync_copy(rows_v, p0.at[sl])

    @pl.when(cid == 1)
    def _():
      pltpu.sync_copy(rows_v, p1.at[sl])

    return carry

  lax.fori_loop(0, RT // K2, out_chunk, 0)


def _message_pass2(y2, src, dst):
  zeros_k = jnp.zeros((K2, 128), jnp.float32)
  f = pl.kernel(
      _msg2_body,
      out_type=(jax.ShapeDtypeStruct((NP, 128), jnp.float32),
                jax.ShapeDtypeStruct((NP, 128), jnp.float32)),
      mesh=_MESH,
      scratch_types=[
          pltpu.VMEM((K2,), jnp.int32),
          pltpu.VMEM((K2,), jnp.int32),
          pltpu.VMEM((K2, 128), jnp.float32),
          pltpu.VMEM_SHARED((NP, 128), jnp.float32),
          pltpu.SemaphoreType.DMA,
      ],
  )
  return f(y2, src, dst, zeros_k)


_RB = 640


def _mm1_body(d0, d1, x_ref, w_ref, ya, yb):
  dinv = lax.rsqrt(d0[...] + d1[...] + 1.0)
  y = jnp.dot(x_ref[...], w_ref[...],
              preferred_element_type=jnp.float32) * dinv
  ya[...] = y[:, :D1 // 2]
  yb[...] = y[:, D1 // 2:]


def _layer1_matmul(deg0, deg1, x, W1):
  grid = (NP // _RB,)
  return pl.pallas_call(
      _mm1_body,
      grid=grid,
      in_specs=[
          pl.BlockSpec((_RB, 1), lambda i: (i, 0)),
          pl.BlockSpec((_RB, 1), lambda i: (i, 0)),
          pl.BlockSpec((_RB, D0), lambda i: (i, 0)),
          pl.BlockSpec((D0, D1), lambda i: (0, 0)),
      ],
      out_specs=(
          pl.BlockSpec((_RB, D1 // 2), lambda i: (i, 0)),
          pl.BlockSpec((_RB, D1 // 2), lambda i: (i, 0)),
      ),
      out_shape=(
          jax.ShapeDtypeStruct((NP, D1 // 2), jnp.float32),
          jax.ShapeDtypeStruct((NP, D1 // 2), jnp.float32),
      ),
  )(deg0, deg1, x, W1)


def _mm2_body(aa, ab, d0, d1, b1, w_ref, y2):
  dinv = lax.rsqrt(d0[...] + d1[...] + 1.0)
  h = jnp.concatenate([aa[...], ab[...]], axis=1)
  h = jnp.maximum(h * dinv + b1[...], 0.0)
  y2[...] = jnp.dot(h, w_ref[...], preferred_element_type=jnp.float32) * dinv


def _layer2_matmul(acc1a, acc1b, deg0, deg1, b1, W2):
  grid = (NP // _RB,)
  return pl.pallas_call(
      _mm2_body,
      grid=grid,
      in_specs=[
          pl.BlockSpec((_RB, D1 // 2), lambda i: (i, 0)),
          pl.BlockSpec((_RB, D1 // 2), lambda i: (i, 0)),
          pl.BlockSpec((_RB, 1), lambda i: (i, 0)),
          pl.BlockSpec((_RB, 1), lambda i: (i, 0)),
          pl.BlockSpec((1, D1), lambda i: (0, 0)),
          pl.BlockSpec((D1, D2), lambda i: (0, 0)),
      ],
      out_specs=pl.BlockSpec((_RB, D2), lambda i: (i, 0)),
      out_shape=jax.ShapeDtypeStruct((NP, D2), jnp.float32),
  )(acc1a, acc1b, deg0, deg1, b1, W2)


def _epi_body(p0, p1, d0, d1, b2, z_ref):
  dinv = lax.rsqrt(d0[...] + d1[...] + 1.0)
  z_ref[...] = (p0[...] + p1[...]) * dinv + b2[...]


def _epilogue(p0, p1, deg0, deg1, b2):
  grid = (NP // _RB,)
  return pl.pallas_call(
      _epi_body,
      grid=grid,
      in_specs=[
          pl.BlockSpec((_RB, D2), lambda i: (i, 0)),
          pl.BlockSpec((_RB, D2), lambda i: (i, 0)),
          pl.BlockSpec((_RB, 1), lambda i: (i, 0)),
          pl.BlockSpec((_RB, 1), lambda i: (i, 0)),
          pl.BlockSpec((1, D2), lambda i: (0, 0)),
      ],
      out_specs=pl.BlockSpec((_RB, D2), lambda i: (i, 0)),
      out_shape=jax.ShapeDtypeStruct((NP, D2), jnp.float32),
  )(p0, p1, deg0, deg1, b2)


@jax.jit
def kernel(x, edge_index, W1, b1, W2, b2):
  src = edge_index[0].astype(jnp.int32)
  dst = edge_index[1].astype(jnp.int32)
  x_pad = jnp.concatenate(
      [x, jnp.zeros((NP - N, D0), jnp.float32)], axis=0)

  h0, h1 = _deg_counts(dst)
  deg0 = h0[:, :1]
  deg1 = h1[:, :1]
  y1a, y1b = _layer1_matmul(deg0, deg1, x_pad, W1)
  acc1a, acc1b = _message_pass1(y1a, y1b, src, dst)
  y2 = _layer2_matmul(acc1a, acc1b, deg0, deg1, b1.reshape(1, D1), W2)
  p0, p1 = _message_pass2(y2, src, dst)
  z = _epilogue(p0, p1, deg0, deg1, b2.reshape(1, D2))
  return z[:N]


# speedup vs baseline: 7.9591x; 1.1263x over previous
import functools
import jax
import jax.numpy as jnp
from jax import lax
from jax.experimental import pallas as pl
from jax.experimental.pallas import tpu as pltpu
from jax.experimental.pallas import tpu_sc as plsc

N = 10000
NP = 10240
E = 160000
EP = 163840
EROWS = EP // 128
D0, D1, D2 = 256, 256, 128
NC, NS = 2, 16
RT = NP // NS
K = 128
CH1 = EP // NS // K
CH2 = EP // (NC * NS) // K

_MESH = plsc.VectorSubcoreMesh(core_axis_name="c", subcore_axis_name="s",
                               num_cores=NC, num_subcores=NS)


def _deg_body(dst2d, zeros_k, ones_k, p0, p1, stage_d, ones_v, acc):
  cid = lax.axis_index("c")
  sid = lax.axis_index("s")
  r0 = sid * RT

  pltpu.sync_copy(zeros_k, ones_v)

  def init_chunk(j, carry):
    pltpu.sync_copy(ones_v, acc.at[pl.ds(r0 + j * K, K)])
    return carry

  lax.fori_loop(0, RT // K, init_chunk, 0)
  pltpu.sync_copy(ones_k, ones_v)
  pltpu.sync_copy(dst2d.at[pl.ds((cid * NS + sid) * CH2, CH2)], stage_d)
  plsc.subcore_barrier()

  def chunk(j, carry):
    pltpu.sync_copy(ones_v, acc.at[stage_d.at[j]], add=True)
    return carry

  lax.fori_loop(0, CH2, chunk, 0)
  plsc.subcore_barrier()

  def out_chunk(j, carry):
    sl = pl.ds(r0 + j * K, K)
    pltpu.sync_copy(acc.at[sl], ones_v)

    @pl.when(cid == 0)
    def _():
      pltpu.sync_copy(ones_v, p0.at[sl])

    @pl.when(cid == 1)
    def _():
      pltpu.sync_copy(ones_v, p1.at[sl])

    return carry

  lax.fori_loop(0, RT // K, out_chunk, 0)


def _deg_counts(dst2d):
  zeros_k = jnp.zeros((K, 128), jnp.float32)
  ones_k = jnp.ones((K, 128), jnp.float32)
  f = pl.kernel(
      _deg_body,
      out_type=(jax.ShapeDtypeStruct((NP, 128), jnp.float32),
                jax.ShapeDtypeStruct((NP, 128), jnp.float32)),
      mesh=_MESH,
      scratch_types=[
          pltpu.VMEM((CH2, 128), jnp.int32),
          pltpu.VMEM((K, 128), jnp.float32),
          pltpu.VMEM_SHARED((NP, 128), jnp.float32),
      ],
  )
  return f(dst2d, zeros_k, ones_k)


def _msg_edge_loop(gather_from, srcp, dstp, e0, svs, dvs, bufs, sems, acc,
                   ch):
  for b in range(2):
    pltpu.sync_copy(srcp.at[pl.ds(e0 + b * K, K)], svs[b])
    pltpu.sync_copy(dstp.at[pl.ds(e0 + b * K, K)], dvs[b])
    pltpu.async_copy(gather_from.at[svs[b]], bufs[b], sems[b])

  def chunk(i, carry):
    j = i * 2
    for b in range(2):
      jb = j + b
      pltpu.make_async_copy(gather_from.at[svs[b]], bufs[b], sems[b]).wait()
      pltpu.sync_copy(bufs[b], acc.at[dvs[b]], add=True)

      @pl.when(jb + 2 < ch)
      def _():
        pltpu.sync_copy(srcp.at[pl.ds(e0 + (jb + 2) * K, K)], svs[b])
        pltpu.sync_copy(dstp.at[pl.ds(e0 + (jb + 2) * K, K)], dvs[b])
        pltpu.async_copy(gather_from.at[svs[b]], bufs[b], sems[b])

    return carry

  lax.fori_loop(0, ch // 2, chunk, 0)


def _msg1_body(ya, yb, srcp, dstp, oa, ob, sv0, sv1, dv0, dv1, buf0, buf1,
               acc, sem0, sem1):
  cid = lax.axis_index("c")
  sid = lax.axis_index("s")
  r0 = sid * RT

  def init_chunk(j, carry):
    sl = pl.ds(r0 + j * K, K)

    @pl.when(cid == 0)
    def _():
      pltpu.sync_copy(ya.at[sl], buf0)

    @pl.when(cid == 1)
    def _():
      pltpu.sync_copy(yb.at[sl], buf0)

    pltpu.sync_copy(buf0, acc.at[sl])
    return carry

  lax.fori_loop(0, RT // K, init_chunk, 0)
  plsc.subcore_barrier()

  e0 = sid * (EP // NS)

  @pl.when(cid == 0)
  def _():
    _msg_edge_loop(ya, srcp, dstp, e0, (sv0, sv1), (dv0, dv1), (buf0, buf1),
                   (sem0, sem1), acc, CH1)

  @pl.when(cid == 1)
  def _():
    _msg_edge_loop(yb, srcp, dstp, e0, (sv0, sv1), (dv0, dv1), (buf0, buf1),
                   (sem0, sem1), acc, CH1)

  plsc.subcore_barrier()

  def out_chunk(j, carry):
    sl = pl.ds(r0 + j * K, K)
    pltpu.sync_copy(acc.at[sl], buf0)

    @pl.when(cid == 0)
    def _():
      pltpu.sync_copy(buf0, oa.at[sl])

    @pl.when(cid == 1)
    def _():
      pltpu.sync_copy(buf0, ob.at[sl])

    return carry

  lax.fori_loop(0, RT // K, out_chunk, 0)


def _message_pass1(ya, yb, srcp, dstp):
  f = pl.kernel(
      _msg1_body,
      out_type=(jax.ShapeDtypeStruct((NP, 128), jnp.float32),
                jax.ShapeDtypeStruct((NP, 128), jnp.float32)),
      mesh=_MESH,
      scratch_types=[
          pltpu.VMEM((K,), jnp.int32),
          pltpu.VMEM((K,), jnp.int32),
          pltpu.VMEM((K,), jnp.int32),
          pltpu.VMEM((K,), jnp.int32),
          pltpu.VMEM((K, 128), jnp.float32),
          pltpu.VMEM((K, 128), jnp.float32),
          pltpu.VMEM_SHARED((NP, 128), jnp.float32),
          pltpu.SemaphoreType.DMA,
          pltpu.SemaphoreType.DMA,
      ],
  )
  return f(ya, yb, srcp, dstp)


def _msg2_body(y2, srcp, dstp, zeros_k, p0, p1, sv0, sv1, dv0, dv1, buf0,
               buf1, acc, sem0, sem1):
  cid = lax.axis_index("c")
  sid = lax.axis_index("s")
  r0 = sid * RT

  @pl.when(cid == 1)
  def _():
    pltpu.sync_copy(zeros_k, buf0)

  def init_chunk(j, carry):
    sl = pl.ds(r0 + j * K, K)

    @pl.when(cid == 0)
    def _():
      pltpu.sync_copy(y2.at[sl], buf0)

    pltpu.sync_copy(buf0, acc.at[sl])
    return carry

  lax.fori_loop(0, RT // K, init_chunk, 0)
  plsc.subcore_barrier()

  e0 = (cid * NS + sid) * (EP // (NC * NS))
  _msg_edge_loop(y2, srcp, dstp, e0, (sv0, sv1), (dv0, dv1), (buf0, buf1),
                 (sem0, sem1), acc, CH2)
  plsc.subcore_barrier()

  def out_chunk(j, carry):
    sl = pl.ds(r0 + j * K, K)
    pltpu.sync_copy(acc.at[sl], buf0)

    @pl.when(cid == 0)
    def _():
      pltpu.sync_copy(buf0, p0.at[sl])

    @pl.when(cid == 1)
    def _():
      pltpu.sync_copy(buf0, p1.at[sl])

    return carry

  lax.fori_loop(0, RT // K, out_chunk, 0)


def _message_pass2(y2, srcp, dstp):
  zeros_k = jnp.zeros((K, 128), jnp.float32)
  f = pl.kernel(
      _msg2_body,
      out_type=(jax.ShapeDtypeStruct((NP, 128), jnp.float32),
                jax.ShapeDtypeStruct((NP, 128), jnp.float32)),
      mesh=_MESH,
      scratch_types=[
          pltpu.VMEM((K,), jnp.int32),
          pltpu.VMEM((K,), jnp.int32),
          pltpu.VMEM((K,), jnp.int32),
          pltpu.VMEM((K,), jnp.int32),
          pltpu.VMEM((K, 128), jnp.float32),
          pltpu.VMEM((K, 128), jnp.float32),
          pltpu.VMEM_SHARED((NP, 128), jnp.float32),
          pltpu.SemaphoreType.DMA,
          pltpu.SemaphoreType.DMA,
      ],
  )
  return f(y2, srcp, dstp, zeros_k)


_RB = 1000


def _mm1_body(d0, d1, x_ref, w_ref, ya, yb):
  dinv = lax.rsqrt(d0[...] + d1[...] + 1.0)
  y = jnp.dot(x_ref[...], w_ref[...],
              preferred_element_type=jnp.float32) * dinv
  ya[...] = y[:, :D1 // 2]
  yb[...] = y[:, D1 // 2:]


def _layer1_matmul(deg0, deg1, x, W1):
  grid = (N // _RB,)
  return pl.pallas_call(
      _mm1_body,
      grid=grid,
      in_specs=[
          pl.BlockSpec((_RB, 1), lambda i: (i, 0)),
          pl.BlockSpec((_RB, 1), lambda i: (i, 0)),
          pl.BlockSpec((_RB, D0), lambda i: (i, 0)),
          pl.BlockSpec((D0, D1), lambda i: (0, 0)),
      ],
      out_specs=(
          pl.BlockSpec((_RB, D1 // 2), lambda i: (i, 0)),
          pl.BlockSpec((_RB, D1 // 2), lambda i: (i, 0)),
      ),
      out_shape=(
          jax.ShapeDtypeStruct((NP, D1 // 2), jnp.float32),
          jax.ShapeDtypeStruct((NP, D1 // 2), jnp.float32),
      ),
  )(deg0, deg1, x, W1)


def _mm2_body(aa, ab, d0, d1, b1, w_ref, y2):
  dinv = lax.rsqrt(d0[...] + d1[...] + 1.0)
  h = jnp.concatenate([aa[...], ab[...]], axis=1)
  h = jnp.maximum(h * dinv + b1[...], 0.0)
  y2[...] = jnp.dot(h, w_ref[...], preferred_element_type=jnp.float32) * dinv


def _layer2_matmul(acc1a, acc1b, deg0, deg1, b1, W2):
  grid = (N // _RB,)
  return pl.pallas_call(
      _mm2_body,
      grid=grid,
      in_specs=[
          pl.BlockSpec((_RB, D1 // 2), lambda i: (i, 0)),
          pl.BlockSpec((_RB, D1 // 2), lambda i: (i, 0)),
          pl.BlockSpec((_RB, 1), lambda i: (i, 0)),
          pl.BlockSpec((_RB, 1), lambda i: (i, 0)),
          pl.BlockSpec((1, D1), lambda i: (0, 0)),
          pl.BlockSpec((D1, D2), lambda i: (0, 0)),
      ],
      out_specs=pl.BlockSpec((_RB, D2), lambda i: (i, 0)),
      out_shape=jax.ShapeDtypeStruct((NP, D2), jnp.float32),
  )(acc1a, acc1b, deg0, deg1, b1, W2)


def _epi_body(p0, p1, d0, d1, b2, z_ref):
  dinv = lax.rsqrt(d0[...] + d1[...] + 1.0)
  z_ref[...] = (p0[...] + p1[...]) * dinv + b2[...]


def _epilogue(p0, p1, deg0, deg1, b2):
  grid = (N // _RB,)
  return pl.pallas_call(
      _epi_body,
      grid=grid,
      in_specs=[
          pl.BlockSpec((_RB, D2), lambda i: (i, 0)),
          pl.BlockSpec((_RB, D2), lambda i: (i, 0)),
          pl.BlockSpec((_RB, 1), lambda i: (i, 0)),
          pl.BlockSpec((_RB, 1), lambda i: (i, 0)),
          pl.BlockSpec((1, D2), lambda i: (0, 0)),
      ],
      out_specs=pl.BlockSpec((_RB, D2), lambda i: (i, 0)),
      out_shape=jax.ShapeDtypeStruct((N, D2), jnp.float32),
  )(p0, p1, deg0, deg1, b2)


@jax.jit
def kernel(x, edge_index, W1, b1, W2, b2):
  pad = jnp.full((EP - E,), N, jnp.int32)
  srcp = jnp.concatenate([edge_index[0].astype(jnp.int32), pad])
  dstp = jnp.concatenate([edge_index[1].astype(jnp.int32), pad])
  dst2d = dstp.reshape(EROWS, 128)

  h0, h1 = _deg_counts(dst2d)
  deg0 = h0[:, :1]
  deg1 = h1[:, :1]
  y1a, y1b = _layer1_matmul(deg0, deg1, x, W1)
  acc1a, acc1b = _message_pass1(y1a, y1b, srcp, dstp)
  y2 = _layer2_matmul(acc1a, acc1b, deg0, deg1, b1.reshape(1, D1), W2)
  p0, p1 = _message_pass2(y2, srcp, dstp)
  return _epilogue(p0, p1, deg0, deg1, b2.reshape(1, D2))
